# Initial kernel scaffold; baseline (speedup 1.0000x reference)
import jax
import jax.numpy as jnp
from jax import lax
from jax.experimental import pallas as pl
from jax.experimental.pallas import tpu as pltpu
from jax.experimental.pallas import tpu_sc as plsc

B, N, D = 64, 576, 768
NUM_REMAIN = 144
NUM_MASKED = N - NUM_REMAIN
NC, NS = 2, 16
NW = NC * NS
B_PER_W = B // NW
CHUNK = 36
N_CHUNKS = NUM_REMAIN // CHUNK
OUT_ROWS = NUM_REMAIN + 1


def _index_body(noise_ref, revert_ref, remain_ref, masked_ref, flat_ref,
                pe_ref):
    b = pl.program_id(0)
    n = noise_ref[0, 0, :]
    col = lax.broadcast_in_dim(n, (N, N), (0,))
    row = lax.broadcast_in_dim(n, (N, N), (1,))
    ii = lax.broadcasted_iota(jnp.int32, (N, N), 0)
    jj = lax.broadcasted_iota(jnp.int32, (N, N), 1)
    lt = (row < col) | ((row == col) & (jj < ii))
    rank = jnp.sum(lt.astype(jnp.int32), axis=1)
    revert_ref[0, 0, :] = rank
    rank_row = lax.broadcast_in_dim(rank, (N, N), (1,))
    shuffle = jnp.sum(jnp.where(rank_row == ii, jj, 0), axis=1)
    remain = shuffle[:NUM_REMAIN]
    remain_ref[0, 0, :] = remain
    masked_ref[0, 0, :] = shuffle[NUM_REMAIN:]
    flat_ref[0, 0, :] = remain + b * N
    pe_ref[0, 0, :] = remain + 1


def _index_call(noise3):
    spec = lambda w: pl.BlockSpec((1, 1, w), lambda b: (b, 0, 0))
    return pl.pallas_call(
        _index_body,
        grid=(B,),
        in_specs=[spec(N)],
        out_specs=[spec(N), spec(NUM_REMAIN), spec(NUM_MASKED),
                   spec(NUM_REMAIN), spec(NUM_REMAIN)],
        out_shape=[
            jax.ShapeDtypeStruct((B, 1, N), jnp.int32),
            jax.ShapeDtypeStruct((B, 1, NUM_REMAIN), jnp.int32),
            jax.ShapeDtypeStruct((B, 1, NUM_MASKED), jnp.int32),
            jax.ShapeDtypeStruct((B, 1, NUM_REMAIN), jnp.int32),
            jax.ShapeDtypeStruct((B, 1, NUM_REMAIN), jnp.int32),
        ],
    )(noise3)


def _gather_body(data_hbm, pe_hbm, fidx_hbm, pidx_hbm, gt_hbm, out_hbm,
                 idx_v, pidx_v, rows_v, pev_v, gt_v, sem):
    c = lax.axis_index("c")
    s = lax.axis_index("s")
    wid = s * NC + c
    pltpu.sync_copy(gt_hbm, gt_v)
    for bb in range(B_PER_W):
        b = wid * B_PER_W + bb
        pltpu.sync_copy(fidx_hbm.at[b], idx_v)
        pltpu.sync_copy(pidx_hbm.at[b], pidx_v)
        pltpu.sync_copy(gt_v, out_hbm.at[pl.ds(b * OUT_ROWS, 1)])
        for ch in range(N_CHUNKS):
            isl = pl.ds(ch * CHUNK, CHUNK)
            pltpu.async_copy(data_hbm.at[idx_v.at[isl]], rows_v, sem).wait()
            pltpu.async_copy(pe_hbm.at[pidx_v.at[isl]], pev_v, sem).wait()

            def add_row(r, carry):
                for d in range(D // 16):
                    sl = pl.ds(d * 16, 16)
                    rows_v[r, sl] = rows_v[r, sl] + pev_v[r, sl]
                return carry

            lax.fori_loop(0, CHUNK, add_row, 0)
            dst = pl.ds(b * OUT_ROWS + 1 + ch * CHUNK, CHUNK)
            pltpu.sync_copy(rows_v, out_hbm.at[dst])


def _gather_call(data2d, pos_enc, flat_idx, pe_idx, gt_row):
    mesh = plsc.VectorSubcoreMesh(core_axis_name="c", subcore_axis_name="s",
                                  num_cores=NC, num_subcores=NS)
    f = pl.kernel(
        _gather_body,
        out_type=jax.ShapeDtypeStruct((B * OUT_ROWS, D), jnp.float32),
        mesh=mesh,
        scratch_types=[
            pltpu.VMEM((NUM_REMAIN,), jnp.int32),
            pltpu.VMEM((NUM_REMAIN,), jnp.int32),
            pltpu.VMEM((CHUNK, D), jnp.float32),
            pltpu.VMEM((CHUNK, D), jnp.float32),
            pltpu.VMEM((1, D), jnp.float32),
            pltpu.SemaphoreType.DMA,
        ],
    )
    return f(data2d, pos_enc, flat_idx, pe_idx, gt_row)


def kernel(data, img_pos_enc, noise, global_token):
    pos_enc = img_pos_enc[:N + 1]
    revert3, remain3, masked3, flat3, pe3 = _index_call(
        noise.reshape(B, 1, N))
    revert_idx = revert3.reshape(B, N)
    remain_idx = remain3.reshape(B, NUM_REMAIN)
    masked_idx = masked3.reshape(B, NUM_MASKED)
    gt_row = global_token + pos_enc[0:1, :]
    out2d = _gather_call(data.reshape(B * N, D), pos_enc,
                         flat3.reshape(B, NUM_REMAIN),
                         pe3.reshape(B, NUM_REMAIN), gt_row)
    out = out2d.reshape(B, OUT_ROWS, D)
    return out, remain_idx, masked_idx, revert_idx


# speedup vs baseline: 3.8147x; 3.8147x over previous
import jax
import jax.numpy as jnp
from jax import lax
from jax.experimental import pallas as pl
from jax.experimental.pallas import tpu as pltpu
from jax.experimental.pallas import tpu_sc as plsc

B, N, D = 64, 576, 768
NUM_REMAIN = 144
NUM_MASKED = N - NUM_REMAIN
NC, NS = 2, 16
NW = NC * NS
B_PER_W = B // NW
CHUNK = 16
N_CHUNKS = NUM_REMAIN // CHUNK
OUT_ROWS = NUM_REMAIN + 1


def _index_body(noise_ref, ncol_ref, revert_ref, shuf_ref):
    n_row = lax.broadcast_in_dim(noise_ref[0, 0, :], (N, N), (1,))
    n_col = lax.broadcast_in_dim(ncol_ref[0, :, 0:1], (N, N), (0, 1))
    jj = lax.broadcasted_iota(jnp.int32, (N, N), 0)
    ii = lax.broadcasted_iota(jnp.int32, (N, N), 1)
    lt = ((n_col < n_row) | ((n_col == n_row) & (jj < ii))).astype(jnp.int32)
    revert_ref[0, 0, :] = jnp.sum(lt, axis=0)
    rank_sub = (N - 1) - jnp.sum(lt, axis=1, keepdims=True)
    e = jnp.broadcast_to(rank_sub, (N, N)) == ii
    shuf_ref[0, 0, :] = jnp.sum(jnp.where(e, jj, 0), axis=0)


def _index_call(noise3, ncol3):
    rspec = pl.BlockSpec((1, 1, N), lambda b: (b, 0, 0))
    cspec = pl.BlockSpec((1, N, 8), lambda b: (b, 0, 0))
    return pl.pallas_call(
        _index_body,
        grid=(B,),
        in_specs=[rspec, cspec],
        out_specs=[rspec, rspec],
        out_shape=[jax.ShapeDtypeStruct((B, 1, N), jnp.int32),
                   jax.ShapeDtypeStruct((B, 1, N), jnp.int32)],
    )(noise3, ncol3)


def _gather_body(data_hbm, pe_hbm, shuf_hbm, gt_hbm, out_hbm,
                 shuf_v, fidx_v, pidx_v, oidx_v, gidx_v, gtbuf_v,
                 rows_v, pev_v, sem_g0, sem_g1, sem_s0, sem_s1):
    c = lax.axis_index("c")
    s = lax.axis_index("s")
    wid = s * NC + c
    iota16 = lax.broadcasted_iota(jnp.int32, (16,), 0)
    gsems = [sem_g0, sem_g1]
    ssems = [sem_s0, sem_s1]

    @pl.when(wid == 0)
    def _():
        gidx_v[pl.ds(0, 16)] = iota16 * 0
        pltpu.async_copy(gt_hbm.at[gidx_v], gtbuf_v, sem_g0).wait()
        for g in range(B // 16):
            gidx_v[pl.ds(0, 16)] = (g * 16 + iota16) * OUT_ROWS
            pltpu.sync_copy(gtbuf_v, out_hbm.at[gidx_v])

    for bb in range(B_PER_W):
        b = wid * B_PER_W + bb
        pltpu.sync_copy(shuf_hbm.at[b], shuf_v)
        base = b * OUT_ROWS + 1
        for ch in range(N_CHUNKS):
            v = shuf_v[0, pl.ds(ch * CHUNK, 16)]
            fidx_v[ch, pl.ds(0, 16)] = v + b * N
            pidx_v[ch, pl.ds(0, 16)] = v + 1
            oidx_v[ch, pl.ds(0, 16)] = base + ch * CHUNK + iota16

        gd = [None, None]
        gp = [None, None]
        sc = [None, None]
        gd[0] = pltpu.async_copy(data_hbm.at[fidx_v.at[0]], rows_v.at[0],
                                 gsems[0])
        gp[0] = pltpu.async_copy(pe_hbm.at[pidx_v.at[0]], pev_v.at[0],
                                 gsems[0])
        for ch in range(N_CHUNKS):
            cur = ch % 2
            nxt = (ch + 1) % 2
            if ch + 1 < N_CHUNKS:
                if sc[nxt] is not None:
                    sc[nxt].wait()
                    sc[nxt] = None
                gd[nxt] = pltpu.async_copy(data_hbm.at[fidx_v.at[ch + 1]],
                                           rows_v.at[nxt], gsems[nxt])
                gp[nxt] = pltpu.async_copy(pe_hbm.at[pidx_v.at[ch + 1]],
                                           pev_v.at[nxt], gsems[nxt])
            gd[cur].wait()
            gp[cur].wait()

            def add_row(r, carry):
                for d in range(D // 16):
                    sl = pl.ds(d * 16, 16)
                    rows_v[cur, r, sl] = rows_v[cur, r, sl] + pev_v[cur, r, sl]
                return carry

            lax.fori_loop(0, CHUNK, add_row, 0)
            if sc[cur] is not None:
                sc[cur].wait()
            sc[cur] = pltpu.async_copy(rows_v.at[cur],
                                       out_hbm.at[oidx_v.at[ch]], ssems[cur])
        for d in sc:
            if d is not None:
                d.wait()


def _gather_call(data2d, pos_enc, shuf3, gt_row):
    mesh = plsc.VectorSubcoreMesh(core_axis_name="c", subcore_axis_name="s",
                                  num_cores=NC, num_subcores=NS)
    f = pl.kernel(
        _gather_body,
        out_type=jax.ShapeDtypeStruct((B * OUT_ROWS, D), jnp.float32),
        mesh=mesh,
        scratch_types=[
            pltpu.VMEM((1, N), jnp.int32),
            pltpu.VMEM((N_CHUNKS, CHUNK), jnp.int32),
            pltpu.VMEM((N_CHUNKS, CHUNK), jnp.int32),
            pltpu.VMEM((N_CHUNKS, CHUNK), jnp.int32),
            pltpu.VMEM((16,), jnp.int32),
            pltpu.VMEM((16, D), jnp.float32),
            pltpu.VMEM((2, CHUNK, D), jnp.float32),
            pltpu.VMEM((2, CHUNK, D), jnp.float32),
            pltpu.SemaphoreType.DMA,
            pltpu.SemaphoreType.DMA,
            pltpu.SemaphoreType.DMA,
            pltpu.SemaphoreType.DMA,
        ],
    )
    return f(data2d, pos_enc, shuf3, gt_row)


def kernel(data, img_pos_enc, noise, global_token):
    pos_enc = img_pos_enc[:N + 1]
    ncol3 = jnp.broadcast_to(noise[:, :, None], (B, N, 8))
    rank3, shuf3 = _index_call(noise.reshape(B, 1, N), ncol3)
    revert_idx = rank3.reshape(B, N)
    shuffle = shuf3.reshape(B, N)
    gt_row = global_token + pos_enc[0:1, :]
    out2d = _gather_call(data.reshape(B * N, D), pos_enc, shuf3, gt_row)
    out = out2d.reshape(B, OUT_ROWS, D)
    return (out, shuffle[:, :NUM_REMAIN], shuffle[:, NUM_REMAIN:],
            revert_idx)
